# Initial kernel scaffold; baseline (speedup 1.0000x reference)
import jax
import jax.numpy as jnp
from jax.experimental import pallas as pl


def kernel(x, weight, bias):
    raise NotImplementedError("write your pallas kernel here")


# speedup vs baseline: 1.2218x; 1.2218x over previous
import functools

import jax
import jax.numpy as jnp
from jax import lax
from jax.experimental import pallas as pl
from jax.experimental.pallas import tpu as pltpu


def _heatmap_block_kernel(x_ref, w_ref, b_ref, o_ref, *, H, W, BB):
    HW = H * W
    taps = lax.dot_general(
        w_ref[...], x_ref[...],
        dimension_numbers=(((1,), (1,)), ((), ())),
        preferred_element_type=jnp.float32)

    idx = lax.broadcasted_iota(jnp.int32, (1, HW), 1)
    row = idx // W
    col = idx - row * W

    heat = jnp.full((BB, HW), b_ref[0, 0], jnp.float32)
    for kh in range(3):
        for kw in range(3):
            dh, dw = kh - 1, kw - 1
            d = dh * W + dw
            tap = taps[kh * 3 + kw]
            if d != 0:
                shifted = pltpu.roll(tap, shift=(-d) % HW, axis=1)
            else:
                shifted = tap
            valid = ((row + dh >= 0) & (row + dh < H)
                     & (col + dw >= 0) & (col + dw < W))
            heat = heat + jnp.where(valid, shifted, 0.0)

    m = jnp.max(heat, axis=1, keepdims=True)
    o_ref[0] = jnp.exp(heat - m)


def kernel(x, weight, bias):
    B, C, H, W = x.shape
    HW = H * W
    BB = 4
    while B % BB:
        BB //= 2

    x_flat = x.reshape(B, C, HW)
    w9 = jnp.transpose(weight[0], (1, 2, 0)).reshape(9, C)
    w16 = jnp.pad(w9, ((0, 16 - 9), (0, 0))).astype(x.dtype)
    b_smem = bias.reshape(1, 1).astype(jnp.float32)

    vmem_limit = 48 << 20

    out = pl.pallas_call(
        functools.partial(_heatmap_block_kernel, H=H, W=W, BB=BB),
        out_shape=jax.ShapeDtypeStruct((B // BB, BB, HW), jnp.float32),
        grid=(B // BB,),
        in_specs=[
            pl.BlockSpec((BB, C, HW), lambda b: (b, 0, 0)),
            pl.BlockSpec((16, C), lambda b: (0, 0)),
            pl.BlockSpec((1, 1), lambda b: (0, 0),
                         memory_space=pltpu.MemorySpace.SMEM),
        ],
        out_specs=pl.BlockSpec((1, BB, HW), lambda b: (b, 0, 0)),
        compiler_params=pltpu.CompilerParams(
            dimension_semantics=("parallel",),
            vmem_limit_bytes=vmem_limit),
    )(x_flat, w16, b_smem)

    return out.reshape(B, H, W)


# speedup vs baseline: 1.3233x; 1.0830x over previous
import functools

import jax
import jax.numpy as jnp
from jax import lax
from jax.experimental import pallas as pl
from jax.experimental.pallas import tpu as pltpu


def _heatmap_block_kernel(x_ref, w_ref, b_ref, o_ref, *, H, W, BB):
    HW = H * W
    idx = lax.broadcasted_iota(jnp.int32, (1, HW), 1)
    row = idx // W
    col = idx - row * W

    for i in range(BB):
        taps = jnp.dot(w_ref[...], x_ref[i],
                       preferred_element_type=jnp.float32)

        heat = jnp.full((1, HW), b_ref[0, 0], jnp.float32)
        for kh in range(3):
            for kw in range(3):
                dh, dw = kh - 1, kw - 1
                d = dh * W + dw
                tap = taps[kh * 3 + kw][None, :]
                if d != 0:
                    shifted = pltpu.roll(tap, shift=(-d) % HW, axis=1)
                else:
                    shifted = tap
                valid = ((row + dh >= 0) & (row + dh < H)
                         & (col + dw >= 0) & (col + dw < W))
                heat = heat + jnp.where(valid, shifted, 0.0)

        m = jnp.max(heat, keepdims=True)
        o_ref[0, i] = jnp.exp(heat - m)[0]


def kernel(x, weight, bias):
    B, C, H, W = x.shape
    HW = H * W
    BB = 4
    while B % BB:
        BB //= 2

    x_flat = x.reshape(B, C, HW)
    w9 = jnp.transpose(weight[0], (1, 2, 0)).reshape(9, C)
    w16 = jnp.pad(w9, ((0, 16 - 9), (0, 0))).astype(x.dtype)
    b_smem = bias.reshape(1, 1).astype(jnp.float32)

    vmem_limit = 48 << 20

    out = pl.pallas_call(
        functools.partial(_heatmap_block_kernel, H=H, W=W, BB=BB),
        out_shape=jax.ShapeDtypeStruct((B // BB, BB, HW), jnp.float32),
        grid=(B // BB,),
        in_specs=[
            pl.BlockSpec((BB, C, HW), lambda b: (b, 0, 0)),
            pl.BlockSpec((16, C), lambda b: (0, 0)),
            pl.BlockSpec((1, 1), lambda b: (0, 0),
                         memory_space=pltpu.MemorySpace.SMEM),
        ],
        out_specs=pl.BlockSpec((1, BB, HW), lambda b: (b, 0, 0)),
        compiler_params=pltpu.CompilerParams(
            dimension_semantics=("parallel",),
            vmem_limit_bytes=vmem_limit),
    )(x_flat, w16, b_smem)

    return out.reshape(B, H, W)


# speedup vs baseline: 4.7123x; 3.5611x over previous
import functools

import jax
import jax.numpy as jnp
from jax import lax
from jax.experimental import pallas as pl
from jax.experimental.pallas import tpu as pltpu


def _heatmap_rows_kernel(x_ref, w_ref, b_ref, o_ref, *, H, W, BB):
    HW = H * W
    P = BB * HW
    taps = lax.dot_general(
        w_ref[...], x_ref[...],
        dimension_numbers=(((1,), (1,)), ((), ())),
        preferred_element_type=jnp.float32)

    idx = lax.broadcasted_iota(jnp.int32, (1, P), 1)
    pi = idx % HW
    row = pi // W
    col = pi - row * W

    heat = jnp.full((1, P), b_ref[0, 0], jnp.float32)
    for kh in range(3):
        for kw in range(3):
            dh, dw = kh - 1, kw - 1
            d = dh * W + dw
            tap = taps[kh * 3 + kw][None, :]
            if d != 0:
                shifted = pltpu.roll(tap, shift=(-d) % P, axis=1)
            else:
                shifted = tap
            valid = ((row + dh >= 0) & (row + dh < H)
                     & (col + dw >= 0) & (col + dw < W))
            heat = heat + jnp.where(valid, shifted, 0.0)

    hb = heat.reshape(BB, HW)
    m = jnp.max(hb, axis=1, keepdims=True)
    o_ref[0] = jnp.exp(hb - m)


def kernel(x, weight, bias):
    B, C, H, W = x.shape
    HW = H * W
    BB = 4
    while B % BB:
        BB //= 2

    x_rows = jnp.transpose(x, (0, 2, 3, 1)).reshape(B * HW, C)
    w9 = jnp.transpose(weight[0], (1, 2, 0)).reshape(9, C)
    w16 = jnp.pad(w9, ((0, 16 - 9), (0, 0))).astype(x.dtype)
    b_smem = bias.reshape(1, 1).astype(jnp.float32)

    out = pl.pallas_call(
        functools.partial(_heatmap_rows_kernel, H=H, W=W, BB=BB),
        out_shape=jax.ShapeDtypeStruct((B // BB, BB, HW), jnp.float32),
        grid=(B // BB,),
        in_specs=[
            pl.BlockSpec((BB * HW, C), lambda b: (b, 0)),
            pl.BlockSpec((16, C), lambda b: (0, 0)),
            pl.BlockSpec((1, 1), lambda b: (0, 0),
                         memory_space=pltpu.MemorySpace.SMEM),
        ],
        out_specs=pl.BlockSpec((1, BB, HW), lambda b: (b, 0, 0)),
        compiler_params=pltpu.CompilerParams(
            dimension_semantics=("parallel",),
            vmem_limit_bytes=48 << 20),
    )(x_rows, w16, b_smem)

    return out.reshape(B, H, W)
